# baseline (speedup 1.0000x reference)
import functools

import jax
import jax.numpy as jnp
from jax import lax
from jax.experimental import pallas as pl
from jax.experimental.pallas import tpu as pltpu
from jax.experimental.pallas import tpu_sc as plsc

N = 10000
E = 320000
D = 128
H = 128
A = 10
G = 8

NC = 2
NS = 16
NW = NC * NS
EPW = E // NW
CH = 50
NCHUNK = EPW // CH
SLOTS = 4
NP = 10240
RPT = NP // NS

_mesh = plsc.VectorSubcoreMesh(
    core_axis_name="c", subcore_axis_name="s", num_cores=NC, num_subcores=NS)


def _seg_body(h_hbm, src_hbm, dst_hbm, z_hbm, p0_hbm, p1_hbm,
              acc, srcI, dstI, rows, semIS, semID, semG, semS):
    c = lax.axis_index("c")
    s = lax.axis_index("s")
    wid = s * NC + c

    pltpu.sync_copy(z_hbm, acc.at[pl.ds(s * RPT, RPT)])
    plsc.subcore_barrier()

    def load_src(j, q):
        pltpu.async_copy(src_hbm.at[wid, j], srcI[q], semIS[q])

    def load_dst(j, q):
        pltpu.async_copy(dst_hbm.at[wid, j], dstI[q], semID[q])

    def wait_src(q):
        pltpu.make_async_copy(src_hbm.at[wid, 0], srcI[q], semIS[q]).wait()

    def wait_dst(q):
        pltpu.make_async_copy(dst_hbm.at[wid, 0], dstI[q], semID[q]).wait()

    def gather(q):
        pltpu.async_copy(h_hbm.at[srcI[q]], rows[q], semG[q])

    def wait_gather(q):
        pltpu.make_async_copy(h_hbm.at[srcI[q]], rows[q], semG[q]).wait()

    def scatter(q):
        pltpu.async_copy(rows[q], acc.at[dstI[q]], semS[q], add=True)

    def wait_scatter(q):
        pltpu.make_async_copy(rows[q], acc.at[dstI[q]], semS[q]).wait()

    for q in range(SLOTS):
        load_src(q, q)
        load_dst(q, q)
    for q in range(SLOTS):
        wait_src(q)
        gather(q)

    def step(k, carry):
        base = SLOTS * k
        for q in range(SLOTS):
            wait_gather(q)
            wait_dst(q)
            scatter(q)
        for q in range(SLOTS):
            wait_scatter(q)
            load_src(base + SLOTS + q, q)
            load_dst(base + SLOTS + q, q)
        for q in range(SLOTS):
            wait_src(q)
            gather(q)
        return carry

    lax.fori_loop(0, NCHUNK // SLOTS - 1, step, 0)
    for q in range(SLOTS):
        wait_gather(q)
        wait_dst(q)
        scatter(q)
    for q in range(SLOTS):
        wait_scatter(q)
    plsc.subcore_barrier()

    orows = pl.ds(s * RPT, RPT)

    @pl.when(c == 0)
    def _():
        pltpu.sync_copy(acc.at[orows], p0_hbm.at[orows])

    @pl.when(c == 1)
    def _():
        pltpu.sync_copy(acc.at[orows], p1_hbm.at[orows])


_seg = pl.kernel(
    _seg_body,
    out_type=[jax.ShapeDtypeStruct((NP, H), jnp.float32),
              jax.ShapeDtypeStruct((NP, H), jnp.float32)],
    mesh=_mesh,
    scratch_types=[
        pltpu.VMEM_SHARED((NP, H), jnp.float32),
        [pltpu.VMEM((CH,), jnp.int32) for _ in range(SLOTS)],
        [pltpu.VMEM((CH,), jnp.int32) for _ in range(SLOTS)],
        [pltpu.VMEM((CH, H), jnp.float32) for _ in range(SLOTS)],
        [pltpu.SemaphoreType.DMA for _ in range(SLOTS)],
        [pltpu.SemaphoreType.DMA for _ in range(SLOTS)],
        [pltpu.SemaphoreType.DMA for _ in range(SLOTS)],
        [pltpu.SemaphoreType.DMA for _ in range(SLOTS)],
    ],
)

def _cnt_body(dst_hbm, z_hbm, one_hbm, c0_hbm, c1_hbm,
              acc, dstI0, dstI1, ones_v, semID0, semID1):
    c = lax.axis_index("c")
    s = lax.axis_index("s")
    wid = s * NC + c
    dstI = (dstI0, dstI1)
    semID = (semID0, semID1)

    pltpu.sync_copy(z_hbm, acc.at[pl.ds(s * RPT, RPT)])
    pltpu.sync_copy(one_hbm, ones_v)
    plsc.subcore_barrier()

    def load_dst(j, p):
        pltpu.async_copy(dst_hbm.at[wid, j], dstI[p], semID[p])

    def wait_dst(p):
        pltpu.make_async_copy(dst_hbm.at[wid, 0], dstI[p], semID[p]).wait()

    load_dst(0, 0)
    load_dst(1, 1)

    def slot_step(nxt, p):
        wait_dst(p)
        pltpu.sync_copy(ones_v, acc.at[dstI[p]], add=True)
        load_dst(nxt, p)

    def step(k, carry):
        slot_step(2 * k + 2, 0)
        slot_step(2 * k + 3, 1)
        return carry

    lax.fori_loop(0, NCHUNK // 2 - 1, step, 0)
    for p in range(2):
        wait_dst(p)
        pltpu.sync_copy(ones_v, acc.at[dstI[p]], add=True)
    plsc.subcore_barrier()

    rows = pl.ds(s * RPT, RPT)

    @pl.when(c == 0)
    def _():
        pltpu.sync_copy(acc.at[rows], c0_hbm.at[rows])

    @pl.when(c == 1)
    def _():
        pltpu.sync_copy(acc.at[rows], c1_hbm.at[rows])


_cnt = pl.kernel(
    _cnt_body,
    out_type=[jax.ShapeDtypeStruct((NP, H), jnp.float32),
              jax.ShapeDtypeStruct((NP, H), jnp.float32)],
    mesh=_mesh,
    scratch_types=[
        pltpu.VMEM_SHARED((NP, H), jnp.float32),
        pltpu.VMEM((CH,), jnp.int32),
        pltpu.VMEM((CH,), jnp.int32),
        pltpu.VMEM((CH, H), jnp.float32),
        pltpu.SemaphoreType.DMA,
        pltpu.SemaphoreType.DMA,
    ],
)

BLK = 1000


def _inv_body(c0, c1, o):
    o[...] = 1.0 / jnp.maximum(c0[:, :16] + c1[:, :16], 1.0)


def _inv_counts(c0, c1):
    row = pl.BlockSpec((BLK, H), lambda i: (i, 0))
    return pl.pallas_call(
        _inv_body,
        grid=(N // BLK,),
        in_specs=[row, row],
        out_specs=pl.BlockSpec((BLK, 16), lambda i: (i, 0)),
        out_shape=jax.ShapeDtypeStruct((NP, 16), jnp.float32),
    )(c0, c1)


def _linr_body(h, wr, bl, o):
    o[...] = (jnp.dot(h[...], wr[...], preferred_element_type=jnp.float32)
              + bl[...])


def _linr(h, Wr, bl):
    row = pl.BlockSpec((BLK, H), lambda i: (i, 0))
    return pl.pallas_call(
        _linr_body,
        grid=(N // BLK,),
        in_specs=[row, pl.BlockSpec((H, H), lambda i: (0, 0)),
                  pl.BlockSpec((1, H), lambda i: (0, 0))],
        out_specs=row,
        out_shape=jax.ShapeDtypeStruct((N, H), jnp.float32),
    )(h, Wr, bl.reshape(1, H))


def _comb_body(relu, p0, p1, inv, wl, r, o):
    mean = (p0[...] + p1[...]) * inv[:, 0:1]
    out = jnp.dot(mean, wl[...], preferred_element_type=jnp.float32) + r[...]
    o[...] = jnp.maximum(out, 0.0) if relu else out


def _dense_layer(p0, p1, inv, Wl, r, relu):
    row = pl.BlockSpec((BLK, H), lambda i: (i, 0))
    return pl.pallas_call(
        functools.partial(_comb_body, relu),
        grid=(N // BLK,),
        in_specs=[row, row, pl.BlockSpec((BLK, 16), lambda i: (i, 0)),
                  pl.BlockSpec((H, H), lambda i: (0, 0)), row],
        out_specs=row,
        out_shape=jax.ShapeDtypeStruct((N, H), jnp.float32),
    )(p0, p1, inv, Wl, r)


def _head_body(h, b, wh, bh, o, accp, accc):
    i = pl.program_id(0)

    @pl.when(i == 0)
    def _():
        accp[...] = jnp.zeros((G, H), jnp.float32)
        accc[...] = jnp.zeros((G, H), jnp.float32)

    bb = b[...].reshape(1, BLK)
    gid = lax.broadcasted_iota(jnp.int32, (G, BLK), 0)
    m = (bb == gid).astype(jnp.float32)
    accp[...] += jnp.dot(m, h[...], preferred_element_type=jnp.float32)
    accc[...] += jnp.sum(m, axis=1, keepdims=True)

    @pl.when(i == pl.num_programs(0) - 1)
    def _():
        pooled = accp[...] / jnp.maximum(accc[...], 1.0)
        o[...] = (jnp.dot(pooled, wh[...], preferred_element_type=jnp.float32)
                  + bh[...])


def _head(h, batch, Wh, bh):
    return pl.pallas_call(
        _head_body,
        grid=(N // BLK,),
        in_specs=[
            pl.BlockSpec((BLK, H), lambda i: (i, 0)),
            pl.BlockSpec((1, 1, BLK), lambda i: (i, 0, 0)),
            pl.BlockSpec((H, A), lambda i: (0, 0)),
            pl.BlockSpec((G, A), lambda i: (0, 0)),
        ],
        out_specs=pl.BlockSpec((G, A), lambda i: (0, 0)),
        out_shape=jax.ShapeDtypeStruct((G, A), jnp.float32),
        scratch_shapes=[pltpu.VMEM((G, H), jnp.float32),
                        pltpu.VMEM((G, H), jnp.float32)],
    )(h, batch.reshape(N // BLK, 1, BLK),
      Wh, jnp.broadcast_to(bh.reshape(1, A), (G, A)))


def kernel(x, edge_index, batch, Wl1, bl1, Wr1, Wl2, bl2, Wr2, Wl3, bl3,
           Wr3, Wl4, bl4, Wr4, Wh, bh):
    src = edge_index[0].astype(jnp.int32).reshape(NW, NCHUNK, CH)
    dst = edge_index[1].astype(jnp.int32).reshape(NW, NCHUNK, CH)
    z = jnp.zeros((RPT, H), jnp.float32)

    c0, c1 = _cnt(dst, z, jnp.ones((CH, H), jnp.float32))
    inv = _inv_counts(c0, c1)

    h = x
    params = [(Wl1, bl1, Wr1), (Wl2, bl2, Wr2), (Wl3, bl3, Wr3),
              (Wl4, bl4, Wr4)]
    for i, (Wl, bl, Wr) in enumerate(params):
        r = _linr(h, Wr, bl)
        p0, p1 = _seg(h, src, dst, z)
        h = _dense_layer(p0, p1, inv, Wl, r, relu=(i < 3))
    return _head(h, batch, Wh, bh)


# speedup vs baseline: 1.2526x; 1.2526x over previous
import functools

import jax
import jax.numpy as jnp
from jax import lax
from jax.experimental import pallas as pl
from jax.experimental.pallas import tpu as pltpu
from jax.experimental.pallas import tpu_sc as plsc

N = 10000
E = 320000
D = 128
H = 128
A = 10
G = 8

NC = 2
NS = 16
NW = NC * NS
EPW = E // NW
CH = 100
NCHUNK = EPW // CH
NP = 10240
RPT = NP // NS

_mesh = plsc.VectorSubcoreMesh(
    core_axis_name="c", subcore_axis_name="s", num_cores=NC, num_subcores=NS)


def _seg_body(h_hbm, src_hbm, dst_hbm, z_hbm, p0_hbm, p1_hbm,
              acc, srcI0, srcI1, dstI0, dstI1, rows0, rows1,
              semIS0, semIS1, semID0, semID1, semG0, semG1):
    c = lax.axis_index("c")
    s = lax.axis_index("s")
    wid = s * NC + c

    srcI = (srcI0, srcI1)
    dstI = (dstI0, dstI1)
    rows = (rows0, rows1)
    semIS = (semIS0, semIS1)
    semID = (semID0, semID1)
    semG = (semG0, semG1)

    pltpu.sync_copy(z_hbm, acc.at[pl.ds(s * RPT, RPT)])
    plsc.subcore_barrier()

    def load_src(j, p):
        pltpu.async_copy(src_hbm.at[wid, j], srcI[p], semIS[p])

    def load_dst(j, p):
        pltpu.async_copy(dst_hbm.at[wid, j], dstI[p], semID[p])

    def wait_src(p):
        pltpu.make_async_copy(src_hbm.at[wid, 0], srcI[p], semIS[p]).wait()

    def wait_dst(p):
        pltpu.make_async_copy(dst_hbm.at[wid, 0], dstI[p], semID[p]).wait()

    def gather(p):
        pltpu.async_copy(h_hbm.at[srcI[p]], rows[p], semG[p])

    def wait_gather(p):
        pltpu.make_async_copy(h_hbm.at[srcI[p]], rows[p], semG[p]).wait()

    def scatter(p):
        pltpu.sync_copy(rows[p], acc.at[dstI[p]], add=True)

    for p in range(2):
        load_src(p, p)
        load_dst(p, p)
    for p in range(2):
        wait_src(p)
        gather(p)

    def slot_step(j, nxt, p):
        wait_gather(p)
        load_src(nxt, p)
        wait_dst(p)
        scatter(p)
        load_dst(nxt, p)
        wait_src(p)
        gather(p)

    def step(k, carry):
        slot_step(2 * k, 2 * k + 2, 0)
        slot_step(2 * k + 1, 2 * k + 3, 1)
        return carry

    lax.fori_loop(0, NCHUNK // 2 - 1, step, 0)
    for p in range(2):
        wait_gather(p)
        wait_dst(p)
        scatter(p)
    plsc.subcore_barrier()

    rows = pl.ds(s * RPT, RPT)

    @pl.when(c == 0)
    def _():
        pltpu.sync_copy(acc.at[rows], p0_hbm.at[rows])

    @pl.when(c == 1)
    def _():
        pltpu.sync_copy(acc.at[rows], p1_hbm.at[rows])


_seg = pl.kernel(
    _seg_body,
    out_type=[jax.ShapeDtypeStruct((NP, H), jnp.float32),
              jax.ShapeDtypeStruct((NP, H), jnp.float32)],
    mesh=_mesh,
    scratch_types=[
        pltpu.VMEM_SHARED((NP, H), jnp.float32),
        pltpu.VMEM((CH,), jnp.int32),
        pltpu.VMEM((CH,), jnp.int32),
        pltpu.VMEM((CH,), jnp.int32),
        pltpu.VMEM((CH,), jnp.int32),
        pltpu.VMEM((CH, H), jnp.float32),
        pltpu.VMEM((CH, H), jnp.float32),
        pltpu.SemaphoreType.DMA,
        pltpu.SemaphoreType.DMA,
        pltpu.SemaphoreType.DMA,
        pltpu.SemaphoreType.DMA,
        pltpu.SemaphoreType.DMA,
        pltpu.SemaphoreType.DMA,
    ],
)

def _cnt_body(dst_hbm, z_hbm, one_hbm, c0_hbm, c1_hbm,
              acc, dstI0, dstI1, ones_v, semID0, semID1):
    c = lax.axis_index("c")
    s = lax.axis_index("s")
    wid = s * NC + c
    dstI = (dstI0, dstI1)
    semID = (semID0, semID1)

    pltpu.sync_copy(z_hbm, acc.at[pl.ds(s * RPT, RPT)])
    pltpu.sync_copy(one_hbm, ones_v)
    plsc.subcore_barrier()

    def load_dst(j, p):
        pltpu.async_copy(dst_hbm.at[wid, j], dstI[p], semID[p])

    def wait_dst(p):
        pltpu.make_async_copy(dst_hbm.at[wid, 0], dstI[p], semID[p]).wait()

    load_dst(0, 0)
    load_dst(1, 1)

    def slot_step(nxt, p):
        wait_dst(p)
        pltpu.sync_copy(ones_v, acc.at[dstI[p]], add=True)
        load_dst(nxt, p)

    def step(k, carry):
        slot_step(2 * k + 2, 0)
        slot_step(2 * k + 3, 1)
        return carry

    lax.fori_loop(0, NCHUNK // 2 - 1, step, 0)
    for p in range(2):
        wait_dst(p)
        pltpu.sync_copy(ones_v, acc.at[dstI[p]], add=True)
    plsc.subcore_barrier()

    rows = pl.ds(s * RPT, RPT)

    @pl.when(c == 0)
    def _():
        pltpu.sync_copy(acc.at[rows], c0_hbm.at[rows])

    @pl.when(c == 1)
    def _():
        pltpu.sync_copy(acc.at[rows], c1_hbm.at[rows])


_cnt = pl.kernel(
    _cnt_body,
    out_type=[jax.ShapeDtypeStruct((NP, H), jnp.float32),
              jax.ShapeDtypeStruct((NP, H), jnp.float32)],
    mesh=_mesh,
    scratch_types=[
        pltpu.VMEM_SHARED((NP, H), jnp.float32),
        pltpu.VMEM((CH,), jnp.int32),
        pltpu.VMEM((CH,), jnp.int32),
        pltpu.VMEM((CH, H), jnp.float32),
        pltpu.SemaphoreType.DMA,
        pltpu.SemaphoreType.DMA,
    ],
)

BLK = 2000


def _inv_body(c0, c1, o):
    o[...] = 1.0 / jnp.maximum(c0[:, :16] + c1[:, :16], 1.0)


def _inv_counts(c0, c1):
    row = pl.BlockSpec((BLK, H), lambda i: (i, 0))
    return pl.pallas_call(
        _inv_body,
        grid=(N // BLK,),
        in_specs=[row, row],
        out_specs=pl.BlockSpec((BLK, 16), lambda i: (i, 0)),
        out_shape=jax.ShapeDtypeStruct((NP, 16), jnp.float32),
    )(c0, c1)


def _dense_body(relu, p0, p1, h, inv, wl, bl, wr, o):
    mean = (p0[...] + p1[...]) * inv[:, 0:1]
    out = (jnp.dot(mean, wl[...], preferred_element_type=jnp.float32)
           + jnp.dot(h[...], wr[...], preferred_element_type=jnp.float32)
           + bl[...])
    o[...] = jnp.maximum(out, 0.0) if relu else out


def _dense_layer(p0, p1, h, inv, Wl, bl, Wr, relu):
    row = pl.BlockSpec((BLK, H), lambda i: (i, 0))
    w = pl.BlockSpec((H, H), lambda i: (0, 0))
    return pl.pallas_call(
        functools.partial(_dense_body, relu),
        grid=(N // BLK,),
        in_specs=[row, row, row, pl.BlockSpec((BLK, 16), lambda i: (i, 0)),
                  w, pl.BlockSpec((1, H), lambda i: (0, 0)), w],
        out_specs=row,
        out_shape=jax.ShapeDtypeStruct((N, H), jnp.float32),
    )(p0, p1, h, inv, Wl, bl.reshape(1, H), Wr)


def _head_body(h, b, wh, bh, o, accp, accc):
    i = pl.program_id(0)

    @pl.when(i == 0)
    def _():
        accp[...] = jnp.zeros((G, H), jnp.float32)
        accc[...] = jnp.zeros((G, H), jnp.float32)

    bb = b[...].reshape(1, BLK)
    gid = lax.broadcasted_iota(jnp.int32, (G, BLK), 0)
    m = (bb == gid).astype(jnp.float32)
    accp[...] += jnp.dot(m, h[...], preferred_element_type=jnp.float32)
    accc[...] += jnp.sum(m, axis=1, keepdims=True)

    @pl.when(i == pl.num_programs(0) - 1)
    def _():
        pooled = accp[...] / jnp.maximum(accc[...], 1.0)
        o[...] = (jnp.dot(pooled, wh[...], preferred_element_type=jnp.float32)
                  + bh[...])


def _head(h, batch, Wh, bh):
    return pl.pallas_call(
        _head_body,
        grid=(N // BLK,),
        in_specs=[
            pl.BlockSpec((BLK, H), lambda i: (i, 0)),
            pl.BlockSpec((1, 1, BLK), lambda i: (i, 0, 0)),
            pl.BlockSpec((H, A), lambda i: (0, 0)),
            pl.BlockSpec((G, A), lambda i: (0, 0)),
        ],
        out_specs=pl.BlockSpec((G, A), lambda i: (0, 0)),
        out_shape=jax.ShapeDtypeStruct((G, A), jnp.float32),
        scratch_shapes=[pltpu.VMEM((G, H), jnp.float32),
                        pltpu.VMEM((G, H), jnp.float32)],
    )(h, batch.reshape(N // BLK, 1, BLK),
      Wh, jnp.broadcast_to(bh.reshape(1, A), (G, A)))


def kernel(x, edge_index, batch, Wl1, bl1, Wr1, Wl2, bl2, Wr2, Wl3, bl3,
           Wr3, Wl4, bl4, Wr4, Wh, bh):
    src = edge_index[0].astype(jnp.int32).reshape(NW, NCHUNK, CH)
    dst = edge_index[1].astype(jnp.int32).reshape(NW, NCHUNK, CH)
    z = jnp.zeros((RPT, H), jnp.float32)

    c0, c1 = _cnt(dst, z, jnp.ones((CH, H), jnp.float32))
    inv = _inv_counts(c0, c1)

    h = x
    params = [(Wl1, bl1, Wr1), (Wl2, bl2, Wr2), (Wl3, bl3, Wr3),
              (Wl4, bl4, Wr4)]
    for i, (Wl, bl, Wr) in enumerate(params):
        p0, p1 = _seg(h, src, dst, z)
        h = _dense_layer(p0, p1, h, inv, Wl, bl, Wr, relu=(i < 3))
    return _head(h, batch, Wh, bh)


# speedup vs baseline: 1.3049x; 1.0418x over previous
import functools

import jax
import jax.numpy as jnp
from jax import lax
from jax.experimental import pallas as pl
from jax.experimental.pallas import tpu as pltpu
from jax.experimental.pallas import tpu_sc as plsc

N = 10000
E = 320000
D = 128
H = 128
A = 10
G = 8

NC = 2
NS = 16
NW = NC * NS
EPW = E // NW
CH = 125
NCHUNK = EPW // CH
NP = 10240
RPT = NP // NS

_mesh = plsc.VectorSubcoreMesh(
    core_axis_name="c", subcore_axis_name="s", num_cores=NC, num_subcores=NS)


def _seg_body(h_hbm, src_hbm, dst_hbm, z_hbm, p0_hbm, p1_hbm,
              acc, srcI0, srcI1, dstI0, dstI1, rows0, rows1,
              semIS0, semIS1, semID0, semID1, semG0, semG1):
    c = lax.axis_index("c")
    s = lax.axis_index("s")
    wid = s * NC + c

    srcI = (srcI0, srcI1)
    dstI = (dstI0, dstI1)
    rows = (rows0, rows1)
    semIS = (semIS0, semIS1)
    semID = (semID0, semID1)
    semG = (semG0, semG1)

    pltpu.sync_copy(z_hbm, acc.at[pl.ds(s * RPT, RPT)])
    plsc.subcore_barrier()

    def load_src(j, p):
        pltpu.async_copy(src_hbm.at[wid, j], srcI[p], semIS[p])

    def load_dst(j, p):
        pltpu.async_copy(dst_hbm.at[wid, j], dstI[p], semID[p])

    def wait_src(p):
        pltpu.make_async_copy(src_hbm.at[wid, 0], srcI[p], semIS[p]).wait()

    def wait_dst(p):
        pltpu.make_async_copy(dst_hbm.at[wid, 0], dstI[p], semID[p]).wait()

    def gather(p):
        pltpu.async_copy(h_hbm.at[srcI[p]], rows[p], semG[p])

    def wait_gather(p):
        pltpu.make_async_copy(h_hbm.at[srcI[p]], rows[p], semG[p]).wait()

    def scatter(p):
        pltpu.sync_copy(rows[p], acc.at[dstI[p]], add=True)

    for p in range(2):
        load_src(p, p)
        load_dst(p, p)
    for p in range(2):
        wait_src(p)
        gather(p)

    def slot_step(j, nxt, p):
        wait_gather(p)
        load_src(nxt, p)
        wait_dst(p)
        scatter(p)
        load_dst(nxt, p)
        wait_src(p)
        gather(p)

    def step(k, carry):
        slot_step(2 * k, 2 * k + 2, 0)
        slot_step(2 * k + 1, 2 * k + 3, 1)
        return carry

    lax.fori_loop(0, NCHUNK // 2 - 1, step, 0)
    for p in range(2):
        wait_gather(p)
        wait_dst(p)
        scatter(p)
    plsc.subcore_barrier()

    rows = pl.ds(s * RPT, RPT)

    @pl.when(c == 0)
    def _():
        pltpu.sync_copy(acc.at[rows], p0_hbm.at[rows])

    @pl.when(c == 1)
    def _():
        pltpu.sync_copy(acc.at[rows], p1_hbm.at[rows])


_seg = pl.kernel(
    _seg_body,
    out_type=[jax.ShapeDtypeStruct((NP, H), jnp.float32),
              jax.ShapeDtypeStruct((NP, H), jnp.float32)],
    mesh=_mesh,
    scratch_types=[
        pltpu.VMEM_SHARED((NP, H), jnp.float32),
        pltpu.VMEM((CH,), jnp.int32),
        pltpu.VMEM((CH,), jnp.int32),
        pltpu.VMEM((CH,), jnp.int32),
        pltpu.VMEM((CH,), jnp.int32),
        pltpu.VMEM((CH, H), jnp.float32),
        pltpu.VMEM((CH, H), jnp.float32),
        pltpu.SemaphoreType.DMA,
        pltpu.SemaphoreType.DMA,
        pltpu.SemaphoreType.DMA,
        pltpu.SemaphoreType.DMA,
        pltpu.SemaphoreType.DMA,
        pltpu.SemaphoreType.DMA,
    ],
)

def _cnt_body(dst_hbm, z_hbm, one_hbm, c0_hbm, c1_hbm,
              acc, dstI0, dstI1, ones_v, semID0, semID1):
    c = lax.axis_index("c")
    s = lax.axis_index("s")
    wid = s * NC + c
    dstI = (dstI0, dstI1)
    semID = (semID0, semID1)

    pltpu.sync_copy(z_hbm, acc.at[pl.ds(s * RPT, RPT)])
    pltpu.sync_copy(one_hbm, ones_v)
    plsc.subcore_barrier()

    def load_dst(j, p):
        pltpu.async_copy(dst_hbm.at[wid, j], dstI[p], semID[p])

    def wait_dst(p):
        pltpu.make_async_copy(dst_hbm.at[wid, 0], dstI[p], semID[p]).wait()

    load_dst(0, 0)
    load_dst(1, 1)

    def slot_step(nxt, p):
        wait_dst(p)
        pltpu.sync_copy(ones_v, acc.at[dstI[p]], add=True)
        load_dst(nxt, p)

    def step(k, carry):
        slot_step(2 * k + 2, 0)
        slot_step(2 * k + 3, 1)
        return carry

    lax.fori_loop(0, NCHUNK // 2 - 1, step, 0)
    for p in range(2):
        wait_dst(p)
        pltpu.sync_copy(ones_v, acc.at[dstI[p]], add=True)
    plsc.subcore_barrier()

    rows = pl.ds(s * RPT, RPT)

    @pl.when(c == 0)
    def _():
        pltpu.sync_copy(acc.at[rows], c0_hbm.at[rows])

    @pl.when(c == 1)
    def _():
        pltpu.sync_copy(acc.at[rows], c1_hbm.at[rows])


_cnt = pl.kernel(
    _cnt_body,
    out_type=[jax.ShapeDtypeStruct((NP, H), jnp.float32),
              jax.ShapeDtypeStruct((NP, H), jnp.float32)],
    mesh=_mesh,
    scratch_types=[
        pltpu.VMEM_SHARED((NP, H), jnp.float32),
        pltpu.VMEM((CH,), jnp.int32),
        pltpu.VMEM((CH,), jnp.int32),
        pltpu.VMEM((CH, H), jnp.float32),
        pltpu.SemaphoreType.DMA,
        pltpu.SemaphoreType.DMA,
    ],
)

BLK = 2000


def _inv_body(c0, c1, o):
    o[...] = 1.0 / jnp.maximum(c0[:, :16] + c1[:, :16], 1.0)


def _inv_counts(c0, c1):
    row = pl.BlockSpec((BLK, H), lambda i: (i, 0))
    return pl.pallas_call(
        _inv_body,
        grid=(N // BLK,),
        in_specs=[row, row],
        out_specs=pl.BlockSpec((BLK, 16), lambda i: (i, 0)),
        out_shape=jax.ShapeDtypeStruct((NP, 16), jnp.float32),
    )(c0, c1)


def _dense_body(relu, p0, p1, h, inv, wl, bl, wr, o):
    mean = (p0[...] + p1[...]) * inv[:, 0:1]
    out = (jnp.dot(mean, wl[...], preferred_element_type=jnp.float32)
           + jnp.dot(h[...], wr[...], preferred_element_type=jnp.float32)
           + bl[...])
    o[...] = jnp.maximum(out, 0.0) if relu else out


def _dense_layer(p0, p1, h, inv, Wl, bl, Wr, relu):
    row = pl.BlockSpec((BLK, H), lambda i: (i, 0))
    w = pl.BlockSpec((H, H), lambda i: (0, 0))
    return pl.pallas_call(
        functools.partial(_dense_body, relu),
        grid=(N // BLK,),
        in_specs=[row, row, row, pl.BlockSpec((BLK, 16), lambda i: (i, 0)),
                  w, pl.BlockSpec((1, H), lambda i: (0, 0)), w],
        out_specs=row,
        out_shape=jax.ShapeDtypeStruct((N, H), jnp.float32),
    )(p0, p1, h, inv, Wl, bl.reshape(1, H), Wr)


def _head_body(h, b, wh, bh, o, accp, accc):
    i = pl.program_id(0)

    @pl.when(i == 0)
    def _():
        accp[...] = jnp.zeros((G, H), jnp.float32)
        accc[...] = jnp.zeros((G, H), jnp.float32)

    bb = b[...].reshape(1, BLK)
    gid = lax.broadcasted_iota(jnp.int32, (G, BLK), 0)
    m = (bb == gid).astype(jnp.float32)
    accp[...] += jnp.dot(m, h[...], preferred_element_type=jnp.float32)
    accc[...] += jnp.sum(m, axis=1, keepdims=True)

    @pl.when(i == pl.num_programs(0) - 1)
    def _():
        pooled = accp[...] / jnp.maximum(accc[...], 1.0)
        o[...] = (jnp.dot(pooled, wh[...], preferred_element_type=jnp.float32)
                  + bh[...])


def _head(h, batch, Wh, bh):
    return pl.pallas_call(
        _head_body,
        grid=(N // BLK,),
        in_specs=[
            pl.BlockSpec((BLK, H), lambda i: (i, 0)),
            pl.BlockSpec((1, 1, BLK), lambda i: (i, 0, 0)),
            pl.BlockSpec((H, A), lambda i: (0, 0)),
            pl.BlockSpec((G, A), lambda i: (0, 0)),
        ],
        out_specs=pl.BlockSpec((G, A), lambda i: (0, 0)),
        out_shape=jax.ShapeDtypeStruct((G, A), jnp.float32),
        scratch_shapes=[pltpu.VMEM((G, H), jnp.float32),
                        pltpu.VMEM((G, H), jnp.float32)],
    )(h, batch.reshape(N // BLK, 1, BLK),
      Wh, jnp.broadcast_to(bh.reshape(1, A), (G, A)))


def kernel(x, edge_index, batch, Wl1, bl1, Wr1, Wl2, bl2, Wr2, Wl3, bl3,
           Wr3, Wl4, bl4, Wr4, Wh, bh):
    src = edge_index[0].astype(jnp.int32).reshape(NW, NCHUNK, CH)
    dst = edge_index[1].astype(jnp.int32).reshape(NW, NCHUNK, CH)
    z = jnp.zeros((RPT, H), jnp.float32)

    c0, c1 = _cnt(dst, z, jnp.ones((CH, H), jnp.float32))
    inv = _inv_counts(c0, c1)

    h = x
    params = [(Wl1, bl1, Wr1), (Wl2, bl2, Wr2), (Wl3, bl3, Wr3),
              (Wl4, bl4, Wr4)]
    for i, (Wl, bl, Wr) in enumerate(params):
        p0, p1 = _seg(h, src, dst, z)
        h = _dense_layer(p0, p1, h, inv, Wl, bl, Wr, relu=(i < 3))
    return _head(h, batch, Wh, bh)


# speedup vs baseline: 1.3197x; 1.0113x over previous
import functools

import jax
import jax.numpy as jnp
from jax import lax
from jax.experimental import pallas as pl
from jax.experimental.pallas import tpu as pltpu
from jax.experimental.pallas import tpu_sc as plsc

N = 10000
E = 320000
D = 128
H = 128
A = 10
G = 8

NC = 2
NS = 16
NW = NC * NS
EPW = E // NW
CH = 125
NCHUNK = EPW // CH
NP = 10240
RPT = NP // NS

_mesh = plsc.VectorSubcoreMesh(
    core_axis_name="c", subcore_axis_name="s", num_cores=NC, num_subcores=NS)


def _seg_body(h_hbm, src_hbm, dst_hbm, z_hbm, p0_hbm, p1_hbm,
              acc, srcI0, srcI1, dstI0, dstI1, rows0, rows1,
              semIS0, semIS1, semID0, semID1, semG0, semG1):
    c = lax.axis_index("c")
    s = lax.axis_index("s")
    wid = s * NC + c

    srcI = (srcI0, srcI1)
    dstI = (dstI0, dstI1)
    rows = (rows0, rows1)
    semIS = (semIS0, semIS1)
    semID = (semID0, semID1)
    semG = (semG0, semG1)

    def load_src(j, p):
        pltpu.async_copy(src_hbm.at[wid, j], srcI[p], semIS[p])

    def load_dst(j, p):
        pltpu.async_copy(dst_hbm.at[wid, j], dstI[p], semID[p])

    def wait_src(p):
        pltpu.make_async_copy(src_hbm.at[wid, 0], srcI[p], semIS[p]).wait()

    def wait_dst(p):
        pltpu.make_async_copy(dst_hbm.at[wid, 0], dstI[p], semID[p]).wait()

    def gather(p):
        pltpu.async_copy(h_hbm.at[srcI[p]], rows[p], semG[p])

    def wait_gather(p):
        pltpu.make_async_copy(h_hbm.at[srcI[p]], rows[p], semG[p]).wait()

    def scatter(p):
        pltpu.sync_copy(rows[p], acc.at[dstI[p]], add=True)

    for p in range(2):
        load_src(p, p)
        load_dst(p, p)
    for p in range(2):
        wait_src(p)
        gather(p)
    pltpu.sync_copy(z_hbm, acc.at[pl.ds(s * RPT, RPT)])
    plsc.subcore_barrier()

    def slot_step(j, nxt, p):
        wait_gather(p)
        load_src(nxt, p)
        wait_dst(p)
        scatter(p)
        load_dst(nxt, p)
        wait_src(p)
        gather(p)

    def step(k, carry):
        slot_step(2 * k, 2 * k + 2, 0)
        slot_step(2 * k + 1, 2 * k + 3, 1)
        return carry

    lax.fori_loop(0, NCHUNK // 2 - 1, step, 0)
    for p in range(2):
        wait_gather(p)
        wait_dst(p)
        scatter(p)
    plsc.subcore_barrier()

    rows = pl.ds(s * RPT, RPT)

    @pl.when(c == 0)
    def _():
        pltpu.sync_copy(acc.at[rows], p0_hbm.at[rows])

    @pl.when(c == 1)
    def _():
        pltpu.sync_copy(acc.at[rows], p1_hbm.at[rows])


_seg = pl.kernel(
    _seg_body,
    out_type=[jax.ShapeDtypeStruct((NP, H), jnp.float32),
              jax.ShapeDtypeStruct((NP, H), jnp.float32)],
    mesh=_mesh,
    scratch_types=[
        pltpu.VMEM_SHARED((NP, H), jnp.float32),
        pltpu.VMEM((CH,), jnp.int32),
        pltpu.VMEM((CH,), jnp.int32),
        pltpu.VMEM((CH,), jnp.int32),
        pltpu.VMEM((CH,), jnp.int32),
        pltpu.VMEM((CH, H), jnp.float32),
        pltpu.VMEM((CH, H), jnp.float32),
        pltpu.SemaphoreType.DMA,
        pltpu.SemaphoreType.DMA,
        pltpu.SemaphoreType.DMA,
        pltpu.SemaphoreType.DMA,
        pltpu.SemaphoreType.DMA,
        pltpu.SemaphoreType.DMA,
    ],
)

def _cnt_body(dst_hbm, z_hbm, one_hbm, c0_hbm, c1_hbm,
              acc, dstI0, dstI1, ones_v, semID0, semID1):
    c = lax.axis_index("c")
    s = lax.axis_index("s")
    wid = s * NC + c
    dstI = (dstI0, dstI1)
    semID = (semID0, semID1)

    def load_dst(j, p):
        pltpu.async_copy(dst_hbm.at[wid, j], dstI[p], semID[p])

    def wait_dst(p):
        pltpu.make_async_copy(dst_hbm.at[wid, 0], dstI[p], semID[p]).wait()

    load_dst(0, 0)
    load_dst(1, 1)
    pltpu.sync_copy(z_hbm, acc.at[pl.ds(s * RPT, RPT)])
    pltpu.sync_copy(one_hbm, ones_v)
    plsc.subcore_barrier()

    def slot_step(nxt, p):
        wait_dst(p)
        pltpu.sync_copy(ones_v, acc.at[dstI[p]], add=True)
        load_dst(nxt, p)

    def step(k, carry):
        slot_step(2 * k + 2, 0)
        slot_step(2 * k + 3, 1)
        return carry

    lax.fori_loop(0, NCHUNK // 2 - 1, step, 0)
    for p in range(2):
        wait_dst(p)
        pltpu.sync_copy(ones_v, acc.at[dstI[p]], add=True)
    plsc.subcore_barrier()

    rows = pl.ds(s * RPT, RPT)

    @pl.when(c == 0)
    def _():
        pltpu.sync_copy(acc.at[rows], c0_hbm.at[rows])

    @pl.when(c == 1)
    def _():
        pltpu.sync_copy(acc.at[rows], c1_hbm.at[rows])


_cnt = pl.kernel(
    _cnt_body,
    out_type=[jax.ShapeDtypeStruct((NP, H), jnp.float32),
              jax.ShapeDtypeStruct((NP, H), jnp.float32)],
    mesh=_mesh,
    scratch_types=[
        pltpu.VMEM_SHARED((NP, H), jnp.float32),
        pltpu.VMEM((CH,), jnp.int32),
        pltpu.VMEM((CH,), jnp.int32),
        pltpu.VMEM((CH, H), jnp.float32),
        pltpu.SemaphoreType.DMA,
        pltpu.SemaphoreType.DMA,
    ],
)

BLK = 2000


def _inv_body(c0, c1, o):
    o[...] = 1.0 / jnp.maximum(c0[:, :16] + c1[:, :16], 1.0)


def _inv_counts(c0, c1):
    row = pl.BlockSpec((BLK, H), lambda i: (i, 0))
    return pl.pallas_call(
        _inv_body,
        grid=(N // BLK,),
        in_specs=[row, row],
        out_specs=pl.BlockSpec((BLK, 16), lambda i: (i, 0)),
        out_shape=jax.ShapeDtypeStruct((NP, 16), jnp.float32),
    )(c0, c1)


def _dense_body(relu, p0, p1, h, inv, wl, bl, wr, o):
    mean = (p0[...] + p1[...]) * inv[:, 0:1]
    out = (jnp.dot(mean, wl[...], preferred_element_type=jnp.float32)
           + jnp.dot(h[...], wr[...], preferred_element_type=jnp.float32)
           + bl[...])
    o[...] = jnp.maximum(out, 0.0) if relu else out


def _dense_layer(p0, p1, h, inv, Wl, bl, Wr, relu):
    row = pl.BlockSpec((BLK, H), lambda i: (i, 0))
    w = pl.BlockSpec((H, H), lambda i: (0, 0))
    return pl.pallas_call(
        functools.partial(_dense_body, relu),
        grid=(N // BLK,),
        in_specs=[row, row, row, pl.BlockSpec((BLK, 16), lambda i: (i, 0)),
                  w, pl.BlockSpec((1, H), lambda i: (0, 0)), w],
        out_specs=row,
        out_shape=jax.ShapeDtypeStruct((N, H), jnp.float32),
    )(p0, p1, h, inv, Wl, bl.reshape(1, H), Wr)


def _head_body(h, b, wh, bh, o, accp, accc):
    i = pl.program_id(0)

    @pl.when(i == 0)
    def _():
        accp[...] = jnp.zeros((G, H), jnp.float32)
        accc[...] = jnp.zeros((G, H), jnp.float32)

    bb = b[...].reshape(1, BLK)
    gid = lax.broadcasted_iota(jnp.int32, (G, BLK), 0)
    m = (bb == gid).astype(jnp.float32)
    accp[...] += jnp.dot(m, h[...], preferred_element_type=jnp.float32)
    accc[...] += jnp.sum(m, axis=1, keepdims=True)

    @pl.when(i == pl.num_programs(0) - 1)
    def _():
        pooled = accp[...] / jnp.maximum(accc[...], 1.0)
        o[...] = (jnp.dot(pooled, wh[...], preferred_element_type=jnp.float32)
                  + bh[...])


def _head(h, batch, Wh, bh):
    return pl.pallas_call(
        _head_body,
        grid=(N // BLK,),
        in_specs=[
            pl.BlockSpec((BLK, H), lambda i: (i, 0)),
            pl.BlockSpec((1, 1, BLK), lambda i: (i, 0, 0)),
            pl.BlockSpec((H, A), lambda i: (0, 0)),
            pl.BlockSpec((G, A), lambda i: (0, 0)),
        ],
        out_specs=pl.BlockSpec((G, A), lambda i: (0, 0)),
        out_shape=jax.ShapeDtypeStruct((G, A), jnp.float32),
        scratch_shapes=[pltpu.VMEM((G, H), jnp.float32),
                        pltpu.VMEM((G, H), jnp.float32)],
    )(h, batch.reshape(N // BLK, 1, BLK),
      Wh, jnp.broadcast_to(bh.reshape(1, A), (G, A)))


def kernel(x, edge_index, batch, Wl1, bl1, Wr1, Wl2, bl2, Wr2, Wl3, bl3,
           Wr3, Wl4, bl4, Wr4, Wh, bh):
    src = edge_index[0].astype(jnp.int32).reshape(NW, NCHUNK, CH)
    dst = edge_index[1].astype(jnp.int32).reshape(NW, NCHUNK, CH)
    z = jnp.zeros((RPT, H), jnp.float32)

    c0, c1 = _cnt(dst, z, jnp.ones((CH, H), jnp.float32))
    inv = _inv_counts(c0, c1)

    h = x
    params = [(Wl1, bl1, Wr1), (Wl2, bl2, Wr2), (Wl3, bl3, Wr3),
              (Wl4, bl4, Wr4)]
    for i, (Wl, bl, Wr) in enumerate(params):
        p0, p1 = _seg(h, src, dst, z)
        h = _dense_layer(p0, p1, h, inv, Wl, bl, Wr, relu=(i < 3))
    return _head(h, batch, Wh, bh)
